# baseline (speedup 1.0000x reference)
import functools

import jax
import jax.numpy as jnp
from jax import lax
from jax.experimental import pallas as pl
from jax.experimental.pallas import tpu as pltpu
from jax.experimental.pallas import tpu_sc as plsc

NUM_FEATS = 9
VOCAB = 100
HIDDEN = 32
HPAIRS = HIDDEN // 2
TABROWS = NUM_FEATS * VOCAB

NC = 2
NS = 16
L = 16
NW = NC * NS

ROWS_PER_W = 2560
CHUNK = 640
NCHUNK = ROWS_PER_W // CHUNK
N_PAD = 102400
N_SC = NW * ROWS_PER_W
M_TC = N_PAD - N_SC
TABP_SZ = TABROWS * HPAIRS

BTC = 256
KC = 1024


def _make_sc_kernel():
  mesh = plsc.VectorSubcoreMesh(
      core_axis_name="c", subcore_axis_name="s",
      num_cores=NC, num_subcores=NS)

  @functools.partial(
      pl.kernel,
      out_type=jax.ShapeDtypeStruct((N_SC * HIDDEN,), jnp.float32),
      mesh=mesh,
      scratch_types=[
          pltpu.VMEM((TABP_SZ,), jnp.int32),
          pltpu.VMEM((CHUNK * NUM_FEATS + L,), jnp.int32),
          pltpu.VMEM((CHUNK * NUM_FEATS + L,), jnp.int32),
          pltpu.VMEM((CHUNK * HIDDEN,), jnp.float32),
          pltpu.VMEM((CHUNK * HIDDEN,), jnp.float32),
          pltpu.SemaphoreType.DMA,
          pltpu.SemaphoreType.DMA,
          pltpu.SemaphoreType.DMA,
          pltpu.SemaphoreType.DMA,
          pltpu.SemaphoreType.DMA,
      ],
      compiler_params=pltpu.CompilerParams(needs_layout_passes=False),
  )
  def sc_kernel(x_hbm, tabp_hbm, out_hbm, tabp_v, idx_v0, idx_v1,
                out_v0, out_v1, sem_tab, sem_i0, sem_i1, sem_o0, sem_o1):
    wid = lax.axis_index("s") * NC + lax.axis_index("c")
    idx_bufs = [idx_v0, idx_v1]
    out_bufs = [out_v0, out_v1]
    sem_i = [sem_i0, sem_i1]
    sem_o = [sem_o0, sem_o1]

    d_tab = pltpu.async_copy(tabp_hbm, tabp_v, sem_tab)

    def start_idx(c):
      row0 = wid * ROWS_PER_W + c * CHUNK
      return pltpu.async_copy(
          x_hbm.at[pl.ds(row0 * NUM_FEATS, CHUNK * NUM_FEATS)],
          idx_bufs[c % 2].at[pl.ds(0, CHUNK * NUM_FEATS)], sem_i[c % 2])

    def start_out(c):
      row0 = wid * ROWS_PER_W + c * CHUNK
      return pltpu.async_copy(
          out_bufs[c % 2],
          out_hbm.at[pl.ds(row0 * HIDDEN, CHUNK * HIDDEN)], sem_o[c % 2])

    d_idx = {0: start_idx(0)}
    d_out = {}

    for c in range(NCHUNK):
      b = c % 2
      if c + 1 < NCHUNK:
        d_idx[c + 1] = start_idx(c + 1)
      d_idx[c].wait()
      if c == 0:
        d_tab.wait()
      if c >= 2:
        d_out[c - 2].wait()

      idx_b = idx_bufs[b]
      out_b = out_bufs[b]

      @plsc.parallel_loop(0, CHUNK, step=1, unroll=8)
      def _(r):
        ibase = r * NUM_FEATS
        xvec = idx_b[pl.ds(ibase, L)]
        bf = []
        for i in range(NUM_FEATS):
          a = xvec[i]
          bf.append(plsc.bitcast(tabp_v[pl.ds(a, HPAIRS)], jnp.bfloat16))
        s01 = bf[0] + bf[1]
        s23 = bf[2] + bf[3]
        s45 = bf[4] + bf[5]
        s67 = bf[6] + bf[7]
        s = ((s01 + s23) + (s45 + s67)) + bf[8]
        lo, hi = plsc.unpack(s, format=plsc.PackFormat.INTERLEAVED)
        obase = r * HIDDEN
        out_b[pl.ds(obase, L)] = lo
        out_b[pl.ds(obase + L, L)] = hi

      d_out[c] = start_out(c)

    d_out[NCHUNK - 2].wait()
    d_out[NCHUNK - 1].wait()

  return sc_kernel


_SC_KERNEL = _make_sc_kernel()


def _tc_body(cidx_ref, tab_ref, out_ref):
  cx = cidx_ref[...]
  iot = lax.broadcasted_iota(jnp.int32, (BTC, KC), 1)
  acc = jnp.zeros((BTC, KC), jnp.bfloat16)
  for i in range(NUM_FEATS):
    acc = acc + (cx[:, i:i + 1] == iot).astype(jnp.bfloat16)
  out_ref[...] = lax.dot_general(
      acc, tab_ref[...], (((1,), (0,)), ((), ())),
      preferred_element_type=jnp.float32)


_TC_KERNEL = pl.pallas_call(
    _tc_body,
    grid=(M_TC // BTC,),
    in_specs=[
        pl.BlockSpec((BTC, NUM_FEATS), lambda m: (m, 0)),
        pl.BlockSpec((KC, HIDDEN), lambda m: (0, 0)),
    ],
    out_specs=pl.BlockSpec((BTC, HIDDEN), lambda m: (m, 0)),
    out_shape=jax.ShapeDtypeStruct((M_TC, HIDDEN), jnp.float32),
)


def _pack_tables(tables):
  tb = tables.astype(jnp.bfloat16)
  ti = lax.bitcast_convert_type(tb, jnp.uint16).astype(jnp.uint32)
  lo16 = ti[..., :HPAIRS]
  hi16 = ti[..., HPAIRS:]
  packed = (hi16 << 16) | lo16
  return lax.bitcast_convert_type(packed, jnp.int32).reshape(-1)


@jax.jit
def kernel(x, tables):
  if x.ndim == 1:
    x = x[:, None]
  n = x.shape[0]
  x = x.astype(jnp.int32)
  feat_off = (jnp.arange(NUM_FEATS, dtype=jnp.int32) * VOCAB)[None, :]
  cidx = x + feat_off
  cp = jnp.pad(cidx, ((0, N_PAD - n), (0, 0)))
  addr = cp * HPAIRS

  tab_bf = jnp.pad(tables.astype(jnp.bfloat16).reshape(TABROWS, HIDDEN),
                   ((0, KC - TABROWS), (0, 0)))

  sc_out = _SC_KERNEL(addr[:N_SC].reshape(-1), _pack_tables(tables))
  tc_out = _TC_KERNEL(cp[N_SC:], tab_bf)
  out = jnp.concatenate(
      [sc_out.reshape(N_SC, HIDDEN), tc_out], axis=0)
  return out[:n]


# speedup vs baseline: 1.4077x; 1.4077x over previous
import functools

import jax
import jax.numpy as jnp
from jax import lax
from jax.experimental import pallas as pl
from jax.experimental.pallas import tpu as pltpu
from jax.experimental.pallas import tpu_sc as plsc

NUM_FEATS = 9
VOCAB = 100
HIDDEN = 32
HPAIRS = HIDDEN // 2

NC = 2
NS = 16
L = 16
NW = NC * NS

ROWS_PER_W = 3200
CHUNK = 640
NCHUNK = ROWS_PER_W // CHUNK
N_PAD = NW * ROWS_PER_W
TABP_SZ = NUM_FEATS * VOCAB * HPAIRS


def _make_sc_kernel():
  mesh = plsc.VectorSubcoreMesh(
      core_axis_name="c", subcore_axis_name="s",
      num_cores=NC, num_subcores=NS)

  @functools.partial(
      pl.kernel,
      out_type=jax.ShapeDtypeStruct((N_PAD * HIDDEN,), jnp.float32),
      mesh=mesh,
      scratch_types=[
          pltpu.VMEM((TABP_SZ,), jnp.int32),
          pltpu.VMEM((CHUNK * NUM_FEATS + L,), jnp.int32),
          pltpu.VMEM((CHUNK * NUM_FEATS + L,), jnp.int32),
          pltpu.VMEM((CHUNK * HIDDEN,), jnp.float32),
          pltpu.VMEM((CHUNK * HIDDEN,), jnp.float32),
          pltpu.SemaphoreType.DMA,
          pltpu.SemaphoreType.DMA,
          pltpu.SemaphoreType.DMA,
          pltpu.SemaphoreType.DMA,
          pltpu.SemaphoreType.DMA,
      ],
      compiler_params=pltpu.CompilerParams(needs_layout_passes=False),
  )
  def sc_kernel(x_hbm, tabp_hbm, out_hbm, tabp_v, idx_v0, idx_v1,
                out_v0, out_v1, sem_tab, sem_i0, sem_i1, sem_o0, sem_o1):
    wid = lax.axis_index("s") * NC + lax.axis_index("c")
    idx_bufs = [idx_v0, idx_v1]
    out_bufs = [out_v0, out_v1]
    sem_i = [sem_i0, sem_i1]
    sem_o = [sem_o0, sem_o1]

    d_tab = pltpu.async_copy(tabp_hbm, tabp_v, sem_tab)

    def start_idx(c):
      row0 = wid * ROWS_PER_W + c * CHUNK
      return pltpu.async_copy(
          x_hbm.at[pl.ds(row0 * NUM_FEATS, CHUNK * NUM_FEATS)],
          idx_bufs[c % 2].at[pl.ds(0, CHUNK * NUM_FEATS)], sem_i[c % 2])

    def start_out(c):
      row0 = wid * ROWS_PER_W + c * CHUNK
      return pltpu.async_copy(
          out_bufs[c % 2],
          out_hbm.at[pl.ds(row0 * HIDDEN, CHUNK * HIDDEN)], sem_o[c % 2])

    d_idx = {0: start_idx(0)}
    d_out = {}

    for c in range(NCHUNK):
      b = c % 2
      if c + 1 < NCHUNK:
        d_idx[c + 1] = start_idx(c + 1)
      d_idx[c].wait()
      if c == 0:
        d_tab.wait()
      if c >= 2:
        d_out[c - 2].wait()

      idx_b = idx_bufs[b]
      out_b = out_bufs[b]

      @plsc.parallel_loop(0, CHUNK, step=L, unroll=1)
      def _(g):
        gbase = g * NUM_FEATS
        ivs = [idx_b[pl.ds(gbase + k * L, L)] for k in range(NUM_FEATS)]
        for m in range(L):
          bf = []
          for i in range(NUM_FEATS):
            t = m * NUM_FEATS + i
            a = ivs[t // L][t % L]
            bf.append(plsc.bitcast(tabp_v[pl.ds(a, HPAIRS)], jnp.bfloat16))
          s01 = bf[0] + bf[1]
          s23 = bf[2] + bf[3]
          s45 = bf[4] + bf[5]
          s67 = bf[6] + bf[7]
          s = ((s01 + s23) + (s45 + s67)) + bf[8]
          lo, hi = plsc.unpack(s, format=plsc.PackFormat.INTERLEAVED)
          obase = (g + m) * HIDDEN
          out_b[pl.ds(obase, L)] = lo
          out_b[pl.ds(obase + L, L)] = hi

      d_out[c] = start_out(c)

    d_out[NCHUNK - 2].wait()
    d_out[NCHUNK - 1].wait()

  return sc_kernel


_SC_KERNEL = _make_sc_kernel()


def _pack_tables(tables):
  tb = tables.astype(jnp.bfloat16)
  ti = lax.bitcast_convert_type(tb, jnp.uint16).astype(jnp.uint32)
  lo16 = ti[..., :HPAIRS]
  hi16 = ti[..., HPAIRS:]
  packed = (hi16 << 16) | lo16
  return lax.bitcast_convert_type(packed, jnp.int32).reshape(-1)


@jax.jit
def kernel(x, tables):
  if x.ndim == 1:
    x = x[:, None]
  n = x.shape[0]
  x = x.astype(jnp.int32)
  feat_off = (jnp.arange(NUM_FEATS, dtype=jnp.int32) * VOCAB)[None, :]
  addr = (x + feat_off) * HPAIRS
  ap = jnp.pad(addr, ((0, N_PAD - n), (0, 0)))
  out_flat = _SC_KERNEL(ap.reshape(-1), _pack_tables(tables))
  return out_flat.reshape(N_PAD, HIDDEN)[:n]


# speedup vs baseline: 1.4535x; 1.0325x over previous
import functools

import jax
import jax.numpy as jnp
from jax import lax
from jax.experimental import pallas as pl
from jax.experimental.pallas import tpu as pltpu
from jax.experimental.pallas import tpu_sc as plsc

NUM_FEATS = 9
VOCAB = 100
HIDDEN = 32
HPAIRS = HIDDEN // 2

NC = 2
NS = 16
L = 16
NW = NC * NS

ROWS_PER_W = 3200
CHUNK = 640
NCHUNK = ROWS_PER_W // CHUNK
N_PAD = NW * ROWS_PER_W
TABP_SZ = NUM_FEATS * VOCAB * HPAIRS


def _make_sc_kernel():
  mesh = plsc.VectorSubcoreMesh(
      core_axis_name="c", subcore_axis_name="s",
      num_cores=NC, num_subcores=NS)

  @functools.partial(
      pl.kernel,
      out_type=jax.ShapeDtypeStruct((N_PAD * HIDDEN,), jnp.float32),
      mesh=mesh,
      scratch_types=[
          pltpu.VMEM((TABP_SZ,), jnp.int32),
          pltpu.VMEM((CHUNK * NUM_FEATS + L,), jnp.int32),
          pltpu.VMEM((CHUNK * NUM_FEATS + L,), jnp.int32),
          pltpu.VMEM((CHUNK * HIDDEN,), jnp.float32),
          pltpu.VMEM((CHUNK * HIDDEN,), jnp.float32),
          pltpu.SemaphoreType.DMA,
          pltpu.SemaphoreType.DMA,
          pltpu.SemaphoreType.DMA,
          pltpu.SemaphoreType.DMA,
          pltpu.SemaphoreType.DMA,
      ],
      compiler_params=pltpu.CompilerParams(needs_layout_passes=False),
  )
  def sc_kernel(x_hbm, tabp_hbm, out_hbm, tabp_v, idx_v0, idx_v1,
                out_v0, out_v1, sem_tab, sem_i0, sem_i1, sem_o0, sem_o1):
    wid = lax.axis_index("s") * NC + lax.axis_index("c")
    idx_bufs = [idx_v0, idx_v1]
    out_bufs = [out_v0, out_v1]
    sem_i = [sem_i0, sem_i1]
    sem_o = [sem_o0, sem_o1]

    d_tab = pltpu.async_copy(tabp_hbm, tabp_v, sem_tab)

    def start_idx(c):
      row0 = wid * ROWS_PER_W + c * CHUNK
      return pltpu.async_copy(
          x_hbm.at[pl.ds(row0 * NUM_FEATS, CHUNK * NUM_FEATS)],
          idx_bufs[c % 2].at[pl.ds(0, CHUNK * NUM_FEATS)], sem_i[c % 2])

    def start_out(c):
      row0 = wid * ROWS_PER_W + c * CHUNK
      return pltpu.async_copy(
          out_bufs[c % 2],
          out_hbm.at[pl.ds(row0 * HIDDEN, CHUNK * HIDDEN)], sem_o[c % 2])

    d_idx = {0: start_idx(0)}
    d_out = {}

    for c in range(NCHUNK):
      b = c % 2
      if c + 1 < NCHUNK:
        d_idx[c + 1] = start_idx(c + 1)
      d_idx[c].wait()
      if c == 0:
        d_tab.wait()
      if c >= 2:
        d_out[c - 2].wait()

      idx_b = idx_bufs[b]
      out_b = out_bufs[b]

      @plsc.parallel_loop(0, CHUNK, step=1, unroll=16)
      def _(r):
        ibase = r * NUM_FEATS
        xvec = idx_b[pl.ds(ibase, L)]
        bf = []
        for i in range(NUM_FEATS):
          a = xvec[i]
          bf.append(plsc.bitcast(tabp_v[pl.ds(a, HPAIRS)], jnp.bfloat16))
        s01 = bf[0] + bf[1]
        s23 = bf[2] + bf[3]
        s45 = bf[4] + bf[5]
        s67 = bf[6] + bf[7]
        s = ((s01 + s23) + (s45 + s67)) + bf[8]
        lo, hi = plsc.unpack(s, format=plsc.PackFormat.INTERLEAVED)
        obase = r * HIDDEN
        out_b[pl.ds(obase, L)] = lo
        out_b[pl.ds(obase + L, L)] = hi

      d_out[c] = start_out(c)

    d_out[NCHUNK - 2].wait()
    d_out[NCHUNK - 1].wait()

  return sc_kernel


_SC_KERNEL = _make_sc_kernel()


def _pack_tables(tables):
  tb = tables.astype(jnp.bfloat16)
  ti = lax.bitcast_convert_type(tb, jnp.uint16).astype(jnp.uint32)
  lo16 = ti[..., :HPAIRS]
  hi16 = ti[..., HPAIRS:]
  packed = (hi16 << 16) | lo16
  return lax.bitcast_convert_type(packed, jnp.int32).reshape(-1)


@jax.jit
def kernel(x, tables):
  if x.ndim == 1:
    x = x[:, None]
  n = x.shape[0]
  x = x.astype(jnp.int32)
  feat_off = (jnp.arange(NUM_FEATS, dtype=jnp.int32) * VOCAB)[None, :]
  addr = (x + feat_off) * HPAIRS
  ap = jnp.pad(addr, ((0, N_PAD - n), (0, 0)))
  out_flat = _SC_KERNEL(ap.reshape(-1), _pack_tables(tables))
  return out_flat.reshape(N_PAD, HIDDEN)[:n]


# speedup vs baseline: 1.4603x; 1.0047x over previous
import functools

import jax
import jax.numpy as jnp
from jax import lax
from jax.experimental import pallas as pl
from jax.experimental.pallas import tpu as pltpu
from jax.experimental.pallas import tpu_sc as plsc

NUM_FEATS = 9
VOCAB = 100
HIDDEN = 32
HPAIRS = HIDDEN // 2

NC = 2
NS = 16
L = 16
NW = NC * NS

ROWS_PER_W = 3200
CHUNK = 640
NCHUNK = ROWS_PER_W // CHUNK
N_PAD = NW * ROWS_PER_W
TABP_SZ = NUM_FEATS * VOCAB * HPAIRS


def _make_sc_kernel():
  mesh = plsc.VectorSubcoreMesh(
      core_axis_name="c", subcore_axis_name="s",
      num_cores=NC, num_subcores=NS)

  @functools.partial(
      pl.kernel,
      out_type=jax.ShapeDtypeStruct((N_PAD * HIDDEN,), jnp.float32),
      mesh=mesh,
      scratch_types=[
          pltpu.VMEM((TABP_SZ,), jnp.int32),
          pltpu.VMEM((CHUNK * NUM_FEATS + L,), jnp.int32),
          pltpu.VMEM((CHUNK * NUM_FEATS + L,), jnp.int32),
          pltpu.VMEM((CHUNK * HIDDEN,), jnp.float32),
          pltpu.VMEM((CHUNK * HIDDEN,), jnp.float32),
          pltpu.SemaphoreType.DMA,
          pltpu.SemaphoreType.DMA,
          pltpu.SemaphoreType.DMA,
          pltpu.SemaphoreType.DMA,
          pltpu.SemaphoreType.DMA,
      ],
      compiler_params=pltpu.CompilerParams(needs_layout_passes=False),
  )
  def sc_kernel(x_hbm, tabp_hbm, out_hbm, tabp_v, idx_v0, idx_v1,
                out_v0, out_v1, sem_tab, sem_i0, sem_i1, sem_o0, sem_o1):
    wid = lax.axis_index("s") * NC + lax.axis_index("c")
    idx_bufs = [idx_v0, idx_v1]
    out_bufs = [out_v0, out_v1]
    sem_i = [sem_i0, sem_i1]
    sem_o = [sem_o0, sem_o1]

    d_tab = pltpu.async_copy(tabp_hbm, tabp_v, sem_tab)

    def start_idx(c):
      row0 = wid * ROWS_PER_W + c * CHUNK
      return pltpu.async_copy(
          x_hbm.at[pl.ds(row0 * NUM_FEATS, CHUNK * NUM_FEATS)],
          idx_bufs[c % 2].at[pl.ds(0, CHUNK * NUM_FEATS)], sem_i[c % 2])

    def start_out(c):
      row0 = wid * ROWS_PER_W + c * CHUNK
      return pltpu.async_copy(
          out_bufs[c % 2],
          out_hbm.at[pl.ds(row0 * HIDDEN, CHUNK * HIDDEN)], sem_o[c % 2])

    d_idx = {0: start_idx(0)}
    d_out = {}

    for c in range(NCHUNK):
      b = c % 2
      if c + 1 < NCHUNK:
        d_idx[c + 1] = start_idx(c + 1)
      d_idx[c].wait()
      if c == 0:
        d_tab.wait()
      if c >= 2:
        d_out[c - 2].wait()

      idx_b = idx_bufs[b]
      out_b = out_bufs[b]

      @plsc.parallel_loop(0, CHUNK, step=1, unroll=8)
      def _(r):
        ibase = r * NUM_FEATS
        xvec = idx_b[pl.ds(ibase, L)]
        bf = []
        for i in range(NUM_FEATS):
          a = xvec[i]
          bf.append(plsc.bitcast(tabp_v[pl.ds(a, HPAIRS)], jnp.bfloat16))
        s01 = bf[0] + bf[1]
        s23 = bf[2] + bf[3]
        s45 = bf[4] + bf[5]
        s67 = bf[6] + bf[7]
        s = ((s01 + s23) + (s45 + s67)) + bf[8]
        lo, hi = plsc.unpack(s, format=plsc.PackFormat.INTERLEAVED)
        obase = r * HIDDEN
        out_b[pl.ds(obase, L)] = lo
        out_b[pl.ds(obase + L, L)] = hi

      d_out[c] = start_out(c)

    d_out[NCHUNK - 2].wait()
    d_out[NCHUNK - 1].wait()

  return sc_kernel


_SC_KERNEL = _make_sc_kernel()


def _pack_tables(tables):
  tb = tables.astype(jnp.bfloat16)
  ti = lax.bitcast_convert_type(tb, jnp.uint16).astype(jnp.uint32)
  lo16 = ti[..., :HPAIRS]
  hi16 = ti[..., HPAIRS:]
  packed = (hi16 << 16) | lo16
  return lax.bitcast_convert_type(packed, jnp.int32).reshape(-1)


@jax.jit
def kernel(x, tables):
  if x.ndim == 1:
    x = x[:, None]
  n = x.shape[0]
  x = x.astype(jnp.int32)
  feat_off = (jnp.arange(NUM_FEATS, dtype=jnp.int32) * VOCAB)[None, :]
  addr = (x + feat_off) * HPAIRS
  ap = jnp.pad(addr, ((0, N_PAD - n), (0, 0)))
  out_flat = _SC_KERNEL(ap.reshape(-1), _pack_tables(tables))
  return out_flat.reshape(N_PAD, HIDDEN)[:n]
